# baseline (speedup 1.0000x reference)
import jax
import jax.numpy as jnp
from jax import lax
from jax.experimental import pallas as pl
from jax.experimental.pallas import tpu as pltpu

E = 64
TOP_K = 2
D_MODEL = 1024
D_FF = 1024
T = 512
BM = 32

_NEG = -3.4e38


def _moe_body(x_ref, gw_ref, gu_ref, dn_ref, out_ref, st_ref, rt_ref, gt_ref, acc_ref):
    e = pl.program_id(0)

    @pl.when(e == 0)
    def _routing():
        x = x_ref[...]
        gw = gw_ref[...]
        logits_t = lax.dot_general(
            gw, x, (((1,), (1,)), ((), ())), preferred_element_type=jnp.float32
        )
        m1 = jnp.max(logits_t, axis=0, keepdims=True)
        s1 = logits_t >= m1
        masked = jnp.where(s1, _NEG, logits_t)
        m2 = jnp.max(masked, axis=0, keepdims=True)
        s2 = (logits_t >= m2) & (~s1)
        w1 = 1.0 / (1.0 + jnp.exp(m2 - m1))
        w2 = 1.0 - w1
        occ = jnp.where(s1 | s2, 1.0, 0.0)
        r_iota = lax.broadcasted_iota(jnp.int32, (T, T), 0)
        c_iota = lax.broadcasted_iota(jnp.int32, (T, T), 1)
        upper = jnp.where(r_iota < c_iota, 1.0, 0.0)
        rt = lax.dot_general(
            occ, upper, (((1,), (0,)), ((), ())), preferred_element_type=jnp.float32
        )
        st_ref[...] = occ
        rt_ref[...] = rt
        gt_ref[...] = jnp.where(s1, w1, 0.0) + jnp.where(s2, w2, 0.0)
        acc_ref[...] = jnp.zeros((T, D_MODEL), jnp.float32)

    srow = st_ref[pl.ds(e, 1), :]
    rrow = rt_ref[pl.ds(e, 1), :]
    grow = gt_ref[pl.ds(e, 1), :]
    n = jnp.sum(srow)
    wgu = gu_ref[0]
    wd = dn_ref[0]

    for b in range(T // BM):

        @pl.when(n > float(BM * b))
        def _block(b=b):
            rr = lax.broadcasted_iota(jnp.int32, (BM, T), 0).astype(
                jnp.float32
            ) + float(BM * b)
            sel = jnp.where((rrow == rr) & (srow > 0.0), 1.0, 0.0)
            xb = lax.dot_general(
                sel, x_ref[...], (((1,), (0,)), ((), ())),
                preferred_element_type=jnp.float32,
            )
            gu = lax.dot_general(
                xb, wgu, (((1,), (1,)), ((), ())), preferred_element_type=jnp.float32
            )
            g = gu[:, :D_FF]
            u = gu[:, D_FF:]
            h = g / (1.0 + jnp.exp(-g)) * u
            eo = lax.dot_general(
                h, wd, (((1,), (1,)), ((), ())), preferred_element_type=jnp.float32
            )
            wcol = lax.dot_general(
                sel, grow, (((1,), (1,)), ((), ())), preferred_element_type=jnp.float32
            )
            acc_ref[...] += lax.dot_general(
                sel, eo * wcol, (((0,), (0,)), ((), ())),
                preferred_element_type=jnp.float32,
            )

    @pl.when(e == E - 1)
    def _emit():
        out_ref[...] = acc_ref[...]


def kernel(hidden_states, gate_weight, gate_up_proj, down_proj):
    return pl.pallas_call(
        _moe_body,
        grid=(E,),
        in_specs=[
            pl.BlockSpec((T, D_MODEL), lambda e: (0, 0)),
            pl.BlockSpec((E, D_MODEL), lambda e: (0, 0)),
            pl.BlockSpec((1, 2 * D_FF, D_MODEL), lambda e: (e, 0, 0)),
            pl.BlockSpec((1, D_MODEL, D_FF), lambda e: (e, 0, 0)),
        ],
        out_specs=pl.BlockSpec((T, D_MODEL), lambda e: (0, 0)),
        out_shape=jax.ShapeDtypeStruct((T, D_MODEL), jnp.float32),
        scratch_shapes=[
            pltpu.VMEM((E, T), jnp.float32),
            pltpu.VMEM((E, T), jnp.float32),
            pltpu.VMEM((E, T), jnp.float32),
            pltpu.VMEM((T, D_MODEL), jnp.float32),
        ],
        compiler_params=pltpu.CompilerParams(
            dimension_semantics=("arbitrary",),
        ),
    )(hidden_states, gate_weight, gate_up_proj, down_proj)
